# baseline (speedup 1.0000x reference)
import functools

import jax
import jax.numpy as jnp
from jax import lax
from jax.experimental import pallas as pl
from jax.experimental.pallas import tpu as pltpu
from jax.experimental.pallas import tpu_sc as plsc

EMBED = 64
SCALE = 8.0
NW = 32
LANES = 16
BBLK = 128
KC = 256


@functools.lru_cache(maxsize=None)
def _build_prep(V):
    KPT = -(-V // (NW * 8)) * 8
    NCH = -(-KPT // KC)
    mesh = plsc.VectorSubcoreMesh(core_axis_name="c", subcore_axis_name="s")

    @functools.partial(
        pl.kernel,
        mesh=mesh,
        out_type=jax.ShapeDtypeStruct((V, EMBED), jnp.float32),
        scratch_types=[
            pltpu.VMEM((EMBED, KC), jnp.float32),
            pltpu.VMEM((EMBED, KC), jnp.float32),
            pltpu.VMEM((KC, EMBED), jnp.float32),
            pltpu.VMEM((KC, EMBED), jnp.float32),
            pltpu.SemaphoreType.DMA,
            pltpu.SemaphoreType.DMA,
        ],
        compiler_params=pltpu.CompilerParams(use_tc_tiling_on_sc=False,
                                             needs_layout_passes=False),
    )
    def prep(lutT_hbm, scr_hbm, a0, a1, b0, b1, gsem, osem):
        wid = lax.axis_index("s") * 2 + lax.axis_index("c")
        base = wid * KPT
        iota = lax.iota(jnp.int32, LANES)
        k_cs = [iota + LANES * kg for kg in range(KC // LANES)]

        def k0_of(j):
            return pl.multiple_of(jnp.minimum(base + j * KC, V - KC), 8)

        def fire_in(j, abuf):
            pltpu.async_copy(lutT_hbm.at[:, pl.ds(k0_of(j), KC)], abuf, gsem)

        def wait_in(abuf):
            pltpu.make_async_copy(lutT_hbm.at[:, pl.ds(0, KC)], abuf,
                                  gsem).wait()

        def drain_out(bbuf):
            pltpu.make_async_copy(scr_hbm.at[pl.ds(0, KC)], bbuf,
                                  osem).wait()

        def transpose_out(j, abuf, bbuf):
            for q in range(EMBED // LANES):
                @plsc.parallel_loop(0, LANES, unroll=2)
                def _(d, _q=q):
                    e_c = ((iota + d) & (LANES - 1)) + (LANES * _q)
                    for kg in range(KC // LANES):
                        v = plsc.load_gather(abuf, [e_c, k_cs[kg]])
                        plsc.store_scatter(bbuf, [k_cs[kg], e_c], v * SCALE)
            pltpu.async_copy(bbuf, scr_hbm.at[pl.ds(k0_of(j), KC)], osem)

        fire_in(0, a0)
        fire_in(1, a1)

        def pair_body(p, carry):
            c0 = 2 * p
            wait_in(a0)

            @pl.when(p >= 1)
            def _():
                drain_out(b0)

            transpose_out(c0, a0, b0)

            @pl.when(c0 + 2 < NCH)
            def _():
                fire_in(c0 + 2, a0)

            wait_in(a1)

            @pl.when(p >= 1)
            def _():
                drain_out(b1)

            transpose_out(c0 + 1, a1, b1)

            @pl.when(c0 + 3 < NCH)
            def _():
                fire_in(c0 + 3, a1)

            return carry

        lax.fori_loop(0, NCH // 2, pair_body, 0)
        if NCH % 2:
            wait_in(a0)
            drain_out(b0)
            transpose_out(NCH - 1, a0, b0)
            drain_out(b1)
            drain_out(b0)
        else:
            drain_out(b0)
            drain_out(b1)

    return prep


@functools.lru_cache(maxsize=None)
def _build_lookup(BATCH, HIST, V):
    assert BATCH == NW * BBLK
    NBH = BATCH // BBLK
    EHI = EMBED // 8

    mesh = plsc.VectorSubcoreMesh(core_axis_name="c", subcore_axis_name="s")

    @functools.partial(
        pl.kernel,
        mesh=mesh,
        out_type=jax.ShapeDtypeStruct((HIST, EHI, NBH, 8, BBLK), jnp.float32),
        scratch_types=[
            pltpu.VMEM((HIST, BBLK), jnp.int32),
            pltpu.VMEM((BBLK, EMBED), jnp.float32),
            pltpu.VMEM((BBLK, EMBED), jnp.float32),
            pltpu.VMEM((EMBED, BBLK), jnp.float32),
            pltpu.VMEM((EMBED, BBLK), jnp.float32),
            pltpu.SemaphoreType.DMA,
            pltpu.SemaphoreType.DMA,
        ],
        compiler_params=pltpu.CompilerParams(use_tc_tiling_on_sc=False,
                                             needs_layout_passes=False),
    )
    def k(xt_hbm, scr_hbm, out_hbm, idx_v, ra, rb, ta, tb, gsem, osem):
        wid = lax.axis_index("s") * 2 + lax.axis_index("c")
        pltpu.sync_copy(xt_hbm.at[:, pl.ds(wid * BBLK, BBLK)], idx_v)
        iota = lax.iota(jnp.int32, LANES)
        r_cs = [iota + LANES * g for g in range(BBLK // LANES)]

        def fire_gather(h, rbuf):
            pltpu.async_copy(scr_hbm.at[idx_v.at[h]], rbuf, gsem)

        def wait_gather(rbuf):
            pltpu.make_async_copy(scr_hbm.at[pl.ds(0, BBLK)], rbuf,
                                  gsem).wait()

        def drain_outs(tbuf):
            pltpu.make_async_copy(scr_hbm.at[pl.ds(0, BBLK)], tbuf,
                                  osem).wait()

        def transpose_store(h, rbuf, tbuf):
            for q in range(EMBED // LANES):
                @plsc.parallel_loop(0, LANES, unroll=2)
                def _(d, _q=q):
                    e_c = ((iota + d) & (LANES - 1)) + (LANES * _q)
                    for g in range(BBLK // LANES):
                        v = plsc.load_gather(rbuf, [r_cs[g], e_c])
                        plsc.store_scatter(tbuf, [e_c, r_cs[g]], v)
            for i in range(EHI):
                pltpu.async_copy(tbuf.at[pl.ds(8 * i, 8)],
                                 out_hbm.at[h, i, wid], osem)

        fire_gather(0, ra)

        def pair_body(p, carry):
            u0 = 2 * p
            wait_gather(ra)
            fire_gather(u0 + 1, rb)

            @pl.when(p >= 1)
            def _():
                drain_outs(ta)

            transpose_store(u0, ra, ta)

            @pl.when(p + 1 < HIST // 2)
            def _():
                fire_gather(u0 + 2, ra)

            @pl.when(p >= 1)
            def _():
                drain_outs(tb)

            wait_gather(rb)
            transpose_store(u0 + 1, rb, tb)
            return carry

        lax.fori_loop(0, HIST // 2, pair_body, 0)
        drain_outs(ta)
        drain_outs(tb)

    return k


def kernel(x, lut):
    BATCH, HIST = x.shape
    V = lut.shape[0]
    xt = jnp.transpose(x).astype(jnp.int32)
    lutT = jnp.transpose(lut)
    scr = _build_prep(V)(lutT)
    out5 = _build_lookup(BATCH, HIST, V)(xt, scr)
    return (out5.transpose(2, 4, 0, 1, 3)
            .reshape(BATCH, HIST, EMBED))
